# baseline (speedup 1.0000x reference)
import functools

import jax
import jax.numpy as jnp
from jax import lax
from jax.experimental import pallas as pl
from jax.experimental.pallas import tpu as pltpu
from jax.experimental.pallas import tpu_sc as plsc

VOCAB = 100000
D = 128
B = 4
T_TEXT = 2048
T_ADD = 512
T_OUT = T_TEXT + T_ADD
N_ROWS = B * T_OUT
NW = 32
WPB = NW // B
TXT_W = T_TEXT // WPB
ADD_W = T_ADD // WPB

_mesh = plsc.VectorSubcoreMesh(core_axis_name="c", subcore_axis_name="s")


@functools.partial(
    pl.kernel,
    out_type=[
        jax.ShapeDtypeStruct((N_ROWS, D), jnp.float32),
        jax.ShapeDtypeStruct((N_ROWS,), jnp.int32),
    ],
    mesh=_mesh,
    scratch_types=[
        pltpu.VMEM((TXT_W,), jnp.int32),
        pltpu.VMEM((ADD_W,), jnp.int32),
        pltpu.VMEM((TXT_W, D), jnp.float32),
        pltpu.VMEM((ADD_W, D), jnp.float32),
        pltpu.VMEM((T_TEXT,), jnp.int32),
        pltpu.VMEM((T_ADD,), jnp.int32),
        pltpu.SemaphoreType.DMA,
        pltpu.SemaphoreType.DMA,
        pltpu.SemaphoreType.DMA,
    ],
)
def _gather_concat(ids_hbm, aids_hbm, am_hbm, addm_hbm, w_hbm,
                   out_emb, out_mask,
                   tidx_v, aidx_v, trows_v, arows_v, mbuf, abuf,
                   isem, sem, msem):
    wid = lax.axis_index("s") * 2 + lax.axis_index("c")
    b = wid // WPB
    j = wid % WPB
    tbase = b * T_OUT + j * TXT_W
    abase = b * T_OUT + T_TEXT + j * ADD_W
    is_mask_worker = wid < B

    pltpu.async_copy(ids_hbm.at[pl.ds(wid * TXT_W, TXT_W)], tidx_v, isem)
    pltpu.async_copy(aids_hbm.at[pl.ds(wid * ADD_W, ADD_W)], aidx_v, isem)

    @pl.when(is_mask_worker)
    def _():
        pltpu.async_copy(am_hbm.at[pl.ds(wid * T_TEXT, T_TEXT)], mbuf, msem)
        pltpu.async_copy(addm_hbm.at[pl.ds(wid * T_ADD, T_ADD)], abuf, msem)

    pltpu.make_async_copy(ids_hbm.at[pl.ds(0, TXT_W)], tidx_v, isem).wait()
    pltpu.make_async_copy(aids_hbm.at[pl.ds(0, ADD_W)], aidx_v, isem).wait()

    gathers = [
        pltpu.async_copy(w_hbm.at[tidx_v.at[pl.ds(0, 128)]],
                         trows_v.at[pl.ds(0, 128)], sem),
        pltpu.async_copy(w_hbm.at[tidx_v.at[pl.ds(128, 128)]],
                         trows_v.at[pl.ds(128, 128)], sem),
        pltpu.async_copy(w_hbm.at[aidx_v], arows_v, sem),
    ]

    @pl.when(is_mask_worker)
    def _():
        pltpu.make_async_copy(am_hbm.at[pl.ds(0, T_TEXT)], mbuf, msem).wait()
        pltpu.make_async_copy(addm_hbm.at[pl.ds(0, T_ADD)], abuf, msem).wait()
        pltpu.async_copy(mbuf, out_mask.at[pl.ds(wid * T_OUT, T_TEXT)], msem)
        pltpu.async_copy(abuf, out_mask.at[pl.ds(wid * T_OUT + T_TEXT, T_ADD)],
                         msem)

    for g in gathers:
        g.wait()

    pltpu.async_copy(trows_v, out_emb.at[pl.ds(tbase, TXT_W)], sem)
    pltpu.async_copy(arows_v, out_emb.at[pl.ds(abase, ADD_W)], sem)
    pltpu.make_async_copy(trows_v, out_emb.at[pl.ds(0, TXT_W)], sem).wait()
    pltpu.make_async_copy(arows_v, out_emb.at[pl.ds(0, ADD_W)], sem).wait()

    @pl.when(is_mask_worker)
    def _():
        pltpu.make_async_copy(mbuf, out_mask.at[pl.ds(0, T_TEXT)],
                              msem).wait()
        pltpu.make_async_copy(abuf, out_mask.at[pl.ds(0, T_ADD)], msem).wait()


def kernel(input_ids, attention_mask, add_ids, add_mask, W):
    emb, mask = _gather_concat(input_ids.reshape(-1), add_ids.reshape(-1),
                               attention_mask.reshape(-1),
                               add_mask.reshape(-1), W)
    return emb.reshape(B, T_OUT, D), mask.reshape(B, T_OUT)


# speedup vs baseline: 1.0554x; 1.0554x over previous
import functools

import jax
import jax.numpy as jnp
from jax import lax
from jax.experimental import pallas as pl
from jax.experimental.pallas import tpu as pltpu
from jax.experimental.pallas import tpu_sc as plsc

VOCAB = 100000
D = 128
B = 4
T_TEXT = 2048
T_ADD = 512
T_OUT = T_TEXT + T_ADD
N_ROWS = B * T_OUT
NW = 32
ROWS_PER_W = N_ROWS // NW
CHUNK = 128
CHUNKS = [(0, CHUNK), (CHUNK, CHUNK), (2 * CHUNK, ROWS_PER_W - 2 * CHUNK)]

_mesh = plsc.VectorSubcoreMesh(core_axis_name="c", subcore_axis_name="s")


@functools.partial(
    pl.kernel,
    out_type=[
        jax.ShapeDtypeStruct((N_ROWS, D), jnp.float32),
        jax.ShapeDtypeStruct((N_ROWS,), jnp.int32),
    ],
    mesh=_mesh,
    scratch_types=[
        pltpu.VMEM((ROWS_PER_W,), jnp.int32),
        pltpu.VMEM((ROWS_PER_W, D), jnp.float32),
        pltpu.VMEM((T_TEXT,), jnp.int32),
        pltpu.VMEM((T_ADD,), jnp.int32),
        pltpu.SemaphoreType.DMA,
        pltpu.SemaphoreType.DMA,
        pltpu.SemaphoreType.DMA,
        pltpu.SemaphoreType.DMA,
        pltpu.SemaphoreType.DMA,
    ],
)
def _gather_concat(ids_hbm, am_hbm, addm_hbm, w_hbm,
                   out_emb, out_mask, idx_v, rows_v, mbuf, abuf,
                   g0sem, g1sem, g2sem, osem, msem):
    wid = lax.axis_index("s") * 2 + lax.axis_index("c")
    base = wid * ROWS_PER_W
    is_mask_worker = wid < B
    gsems = [g0sem, g1sem, g2sem]

    pltpu.sync_copy(ids_hbm.at[pl.ds(base, ROWS_PER_W)], idx_v)

    @pl.when(is_mask_worker)
    def _():
        pltpu.async_copy(am_hbm.at[pl.ds(wid * T_TEXT, T_TEXT)], mbuf, msem)
        pltpu.async_copy(addm_hbm.at[pl.ds(wid * T_ADD, T_ADD)], abuf, msem)

    gathers = [
        pltpu.async_copy(w_hbm.at[idx_v.at[pl.ds(off, n)]],
                         rows_v.at[pl.ds(off, n)], gsems[j])
        for j, (off, n) in enumerate(CHUNKS)
    ]

    @pl.when(is_mask_worker)
    def _():
        pltpu.make_async_copy(am_hbm.at[pl.ds(0, T_TEXT)], mbuf, msem).wait()
        pltpu.make_async_copy(addm_hbm.at[pl.ds(0, T_ADD)], abuf, msem).wait()
        pltpu.async_copy(mbuf, out_mask.at[pl.ds(wid * T_OUT, T_TEXT)], msem)
        pltpu.async_copy(abuf, out_mask.at[pl.ds(wid * T_OUT + T_TEXT, T_ADD)],
                         msem)

    for j, (off, n) in enumerate(CHUNKS):
        gathers[j].wait()
        pltpu.async_copy(rows_v.at[pl.ds(off, n)],
                         out_emb.at[pl.ds(base + off, n)], osem)
    for off, n in CHUNKS:
        pltpu.make_async_copy(rows_v.at[pl.ds(off, n)],
                              out_emb.at[pl.ds(base + off, n)], osem).wait()

    @pl.when(is_mask_worker)
    def _():
        pltpu.make_async_copy(mbuf, out_mask.at[pl.ds(0, T_TEXT)],
                              msem).wait()
        pltpu.make_async_copy(abuf, out_mask.at[pl.ds(0, T_ADD)], msem).wait()


def kernel(input_ids, attention_mask, add_ids, add_mask, W):
    ids = jnp.concatenate([input_ids, add_ids], axis=1).reshape(-1)
    emb, mask = _gather_concat(ids, attention_mask.reshape(-1),
                               add_mask.reshape(-1), W)
    return emb.reshape(B, T_OUT, D), mask.reshape(B, T_OUT)
